# baseline (speedup 1.0000x reference)
import jax
import jax.numpy as jnp
from jax import lax
from jax.experimental import pallas as pl
from jax.experimental.pallas import tpu as pltpu
from jax.experimental.pallas import tpu_sc as plsc

B, D, S = 4, 2048, 4096
D_BLK = 128
S_BLK = 128
BG = 2
N_D_CHUNKS = D // D_BLK
N_S_TILES = S // S_BLK // 2
N_VEC = S_BLK // 16
N_ITEMS = N_S_TILES * (B // BG)


def _sc_body(x_hbm, pos_hbm, scale_hbm, out_hbm,
             pos_v0, pos_v1, post_v, x_v0, x_v1, scale_v,
             ldp0, ldp1, ldx0, ldx1, st0, st1):
    pos_bufs = (pos_v0, pos_v1)
    x_bufs = (x_v0, x_v1)
    ldp = (ldp0, ldp1)
    ldx = (ldx0, ldx1)
    st = (st0, st1)

    wid = lax.axis_index("s") * 2 + lax.axis_index("c")
    dchunk = lax.rem(wid, N_D_CHUNKS)
    sgroup = wid // N_D_CHUNKS
    d0 = dchunk * D_BLK
    s_base = sgroup * N_S_TILES * S_BLK
    pltpu.sync_copy(scale_hbm.at[pl.ds(d0, D_BLK)], scale_v)
    iota = lax.iota(jnp.int32, 16)
    c16 = jnp.full((16,), 16, jnp.int32)

    def x_slice(t, g):
        s0 = s_base + t * S_BLK
        return (pl.ds(BG * g, BG), pl.ds(d0, D_BLK), pl.ds(s0, S_BLK))

    def start_loads(t, g, xslot, pslot, with_pos):
        if with_pos:
            s0 = s_base + t * S_BLK
            pltpu.make_async_copy(
                pos_hbm.at[pl.ds(s0, S_BLK), pl.ds(d0, D_BLK)],
                pos_bufs[pslot], ldp[pslot]).start()
        pltpu.make_async_copy(
            x_hbm.at[x_slice(t, g)], x_bufs[xslot], ldx[xslot]).start()

    def wait_loads(t, g, xslot, pslot, with_pos):
        if with_pos:
            s0 = s_base + t * S_BLK
            pltpu.make_async_copy(
                pos_hbm.at[pl.ds(s0, S_BLK), pl.ds(d0, D_BLK)],
                pos_bufs[pslot], ldp[pslot]).wait()
        pltpu.make_async_copy(
            x_hbm.at[x_slice(t, g)], x_bufs[xslot], ldx[xslot]).wait()

    def start_store(t, g, xslot):
        pltpu.make_async_copy(
            x_bufs[xslot], out_hbm.at[x_slice(t, g)], st[xslot]).start()

    def wait_store(t, g, xslot):
        pltpu.make_async_copy(
            x_bufs[xslot], out_hbm.at[x_slice(t, g)], st[xslot]).wait()

    def transpose_pos(pslot):
        pos_ref = pos_bufs[pslot]

        def d16_body(d16, carry):
            base_d = d16 * 16
            scv = scale_v[pl.ds(base_d, 16)]
            dlane = iota + jnp.full((16,), base_d, jnp.int32)

            @plsc.parallel_loop(0, 16, 1, unroll=4)
            def h_body(h):
                rot = lax.rem(iota + jnp.full((16,), h, jnp.int32), c16)
                for j in range(N_VEC):
                    srow = rot + jnp.full((16,), j * 16, jnp.int32)
                    pv = plsc.load_gather(pos_ref, [srow, dlane])
                    plsc.store_scatter(post_v, [dlane, srow], pv * scv)

            return carry

        lax.fori_loop(0, D_BLK // 16, d16_body, 0)

    def add_pass(xslot):
        x_ref = x_bufs[xslot]

        @plsc.parallel_loop(0, D_BLK, 1, unroll=8)
        def d_row(d):
            prow = [post_v[d, pl.ds(16 * j, 16)] for j in range(N_VEC)]
            for bb in range(BG):
                for j in range(N_VEC):
                    sl = pl.ds(16 * j, 16)
                    x_ref[bb, d, sl] = x_ref[bb, d, sl] + prow[j]

    start_loads(0, 0, 0, 0, True)

    def step(k, carry):
        for r in range(4):
            t = 2 * k + (r // 2)
            g = r % 2
            xs = r % 2
            ps = r // 2
            if r == 0:
                @pl.when(k > 0)
                def _():
                    wait_store(2 * k - 1, 1, 1)
            else:
                pt = 2 * k + ((r - 1) // 2)
                wait_store(pt, (r - 1) % 2, (r - 1) % 2)
            if r < 3:
                nt = 2 * k + ((r + 1) // 2)
                start_loads(nt, (r + 1) % 2, (r + 1) % 2, (r + 1) // 2,
                            with_pos=((r + 1) % 2 == 0))
            else:
                @pl.when(k < N_ITEMS // 4 - 1)
                def _():
                    start_loads(2 * k + 2, 0, 0, 0, True)
            wait_loads(t, g, xs, ps, with_pos=(g == 0))
            if g == 0:
                transpose_pos(ps)
            add_pass(xs)
            start_store(t, g, xs)
        return carry

    lax.fori_loop(0, N_ITEMS // 4, step, 0)
    wait_store(2 * (N_ITEMS // 4) - 1, 1, 1)


def kernel(x, adj_inp, cheb_polynomials, L_tilde, pos_table, scale):
    x3 = x.reshape(B, D, S)
    scale1 = scale.reshape(D)
    mesh = plsc.VectorSubcoreMesh(core_axis_name="c", subcore_axis_name="s")
    run = pl.kernel(
        _sc_body,
        mesh=mesh,
        compiler_params=pltpu.CompilerParams(needs_layout_passes=False),
        out_type=jax.ShapeDtypeStruct((B, D, S), jnp.float32),
        scratch_types=[
            pltpu.VMEM((S_BLK, D_BLK), jnp.float32),
            pltpu.VMEM((S_BLK, D_BLK), jnp.float32),
            pltpu.VMEM((D_BLK, S_BLK), jnp.float32),
            pltpu.VMEM((BG, D_BLK, S_BLK), jnp.float32),
            pltpu.VMEM((BG, D_BLK, S_BLK), jnp.float32),
            pltpu.VMEM((D_BLK,), jnp.float32),
            pltpu.SemaphoreType.DMA,
            pltpu.SemaphoreType.DMA,
            pltpu.SemaphoreType.DMA,
            pltpu.SemaphoreType.DMA,
            pltpu.SemaphoreType.DMA,
            pltpu.SemaphoreType.DMA,
        ],
    )
    out = run(x3, pos_table, scale1)
    return out.reshape(B, D, S, 1)


# speedup vs baseline: 1.0389x; 1.0389x over previous
import jax
import jax.numpy as jnp
from jax import lax
from jax.experimental import pallas as pl
from jax.experimental.pallas import tpu as pltpu
from jax.experimental.pallas import tpu_sc as plsc

B, D, S = 4, 2048, 4096
D_BLK = 128
S_BLK = 128
BG = 2
N_D_CHUNKS = D // D_BLK
N_S_TILES = S // S_BLK // 2
N_VEC = S_BLK // 16
N_ITEMS = N_S_TILES * (B // BG)


def _sc_body(x_hbm, pos_hbm, scale_hbm, out_hbm,
             pos_v0, pos_v1, post_v, x_v0, x_v1, scale_v,
             ldp0, ldp1, ldx0, ldx1, st0, st1):
    pos_bufs = (pos_v0, pos_v1)
    x_bufs = (x_v0, x_v1)
    ldp = (ldp0, ldp1)
    ldx = (ldx0, ldx1)
    st = (st0, st1)

    wid = lax.axis_index("s") * 2 + lax.axis_index("c")
    dchunk = lax.rem(wid, N_D_CHUNKS)
    sgroup = wid // N_D_CHUNKS
    d0 = dchunk * D_BLK
    s_base = sgroup * N_S_TILES * S_BLK
    pltpu.sync_copy(scale_hbm.at[pl.ds(d0, D_BLK)], scale_v)
    iota = lax.iota(jnp.int32, 16)
    c16 = jnp.full((16,), 16, jnp.int32)

    def x_slice(t, g):
        s0 = s_base + t * S_BLK
        return (pl.ds(BG * g, BG), pl.ds(d0, D_BLK), pl.ds(s0, S_BLK))

    def start_loads(t, g, xslot, pslot, with_pos):
        if with_pos:
            s0 = s_base + t * S_BLK
            pltpu.make_async_copy(
                pos_hbm.at[pl.ds(s0, S_BLK), pl.ds(d0, D_BLK)],
                pos_bufs[pslot], ldp[pslot]).start()
        pltpu.make_async_copy(
            x_hbm.at[x_slice(t, g)], x_bufs[xslot], ldx[xslot]).start()

    def wait_loads(t, g, xslot, pslot, with_pos):
        if with_pos:
            s0 = s_base + t * S_BLK
            pltpu.make_async_copy(
                pos_hbm.at[pl.ds(s0, S_BLK), pl.ds(d0, D_BLK)],
                pos_bufs[pslot], ldp[pslot]).wait()
        pltpu.make_async_copy(
            x_hbm.at[x_slice(t, g)], x_bufs[xslot], ldx[xslot]).wait()

    def start_store(t, g, xslot):
        pltpu.make_async_copy(
            x_bufs[xslot], out_hbm.at[x_slice(t, g)], st[xslot]).start()

    def wait_store(t, g, xslot):
        pltpu.make_async_copy(
            x_bufs[xslot], out_hbm.at[x_slice(t, g)], st[xslot]).wait()

    def transpose_pos(pslot):
        pos_ref = pos_bufs[pslot]

        def d16_body(d16, carry):
            base_d = d16 * 16
            scv = scale_v[pl.ds(base_d, 16)]
            dlane = iota + jnp.full((16,), base_d, jnp.int32)

            @plsc.parallel_loop(0, 16, 1, unroll=2)
            def h_body(h):
                rot = lax.rem(iota + jnp.full((16,), h, jnp.int32), c16)
                for j in range(N_VEC):
                    srow = rot + jnp.full((16,), j * 16, jnp.int32)
                    pv = plsc.load_gather(pos_ref, [srow, dlane])
                    plsc.store_scatter(post_v, [dlane, srow], pv * scv)

            return carry

        lax.fori_loop(0, D_BLK // 16, d16_body, 0)

    def add_pass(xslot):
        x_ref = x_bufs[xslot]

        @plsc.parallel_loop(0, D_BLK, 1, unroll=4)
        def d_row(d):
            prow = [post_v[d, pl.ds(16 * j, 16)] for j in range(N_VEC)]
            for bb in range(BG):
                for j in range(N_VEC):
                    sl = pl.ds(16 * j, 16)
                    x_ref[bb, d, sl] = x_ref[bb, d, sl] + prow[j]

    start_loads(0, 0, 0, 0, True)

    def step(k, carry):
        for r in range(4):
            t = 2 * k + (r // 2)
            g = r % 2
            xs = r % 2
            ps = r // 2
            if r == 0:
                @pl.when(k > 0)
                def _():
                    wait_store(2 * k - 1, 1, 1)
            else:
                pt = 2 * k + ((r - 1) // 2)
                wait_store(pt, (r - 1) % 2, (r - 1) % 2)
            if r < 3:
                nt = 2 * k + ((r + 1) // 2)
                start_loads(nt, (r + 1) % 2, (r + 1) % 2, (r + 1) // 2,
                            with_pos=((r + 1) % 2 == 0))
            else:
                @pl.when(k < N_ITEMS // 4 - 1)
                def _():
                    start_loads(2 * k + 2, 0, 0, 0, True)
            wait_loads(t, g, xs, ps, with_pos=(g == 0))
            if g == 0:
                transpose_pos(ps)
            add_pass(xs)
            start_store(t, g, xs)
        return carry

    lax.fori_loop(0, N_ITEMS // 4, step, 0)
    wait_store(2 * (N_ITEMS // 4) - 1, 1, 1)


def kernel(x, adj_inp, cheb_polynomials, L_tilde, pos_table, scale):
    x3 = x.reshape(B, D, S)
    scale1 = scale.reshape(D)
    mesh = plsc.VectorSubcoreMesh(core_axis_name="c", subcore_axis_name="s")
    run = pl.kernel(
        _sc_body,
        mesh=mesh,
        compiler_params=pltpu.CompilerParams(needs_layout_passes=False),
        out_type=jax.ShapeDtypeStruct((B, D, S), jnp.float32),
        scratch_types=[
            pltpu.VMEM((S_BLK, D_BLK), jnp.float32),
            pltpu.VMEM((S_BLK, D_BLK), jnp.float32),
            pltpu.VMEM((D_BLK, S_BLK), jnp.float32),
            pltpu.VMEM((BG, D_BLK, S_BLK), jnp.float32),
            pltpu.VMEM((BG, D_BLK, S_BLK), jnp.float32),
            pltpu.VMEM((D_BLK,), jnp.float32),
            pltpu.SemaphoreType.DMA,
            pltpu.SemaphoreType.DMA,
            pltpu.SemaphoreType.DMA,
            pltpu.SemaphoreType.DMA,
            pltpu.SemaphoreType.DMA,
            pltpu.SemaphoreType.DMA,
        ],
    )
    out = run(x3, pos_table, scale1)
    return out.reshape(B, D, S, 1)
